# baseline (speedup 1.0000x reference)
import functools

import jax
import jax.numpy as jnp
from jax import lax
from jax.experimental import pallas as pl
from jax.experimental.pallas import tpu as pltpu
from jax.experimental.pallas import tpu_sc as plsc

MU = 0.02
BINS = 10
NC = 2
NS = 16
L = 16
NW = NC * NS

N_TOTAL = 2000000 * 4
PER_W = N_TOTAL // NW
CHUNK = 50000
N_CHUNKS = PER_W // CHUNK
VECS = CHUNK // L
UNROLL = 5
TAB_STRIDE = 256


def _sc_histogram(d_flat):
    mesh = plsc.VectorSubcoreMesh(
        core_axis_name="c", subcore_axis_name="s", num_cores=NC, num_subcores=NS
    )

    @functools.partial(
        pl.kernel,
        out_type=(
            jax.ShapeDtypeStruct((NW, BINS * L), jnp.float32),
            jax.ShapeDtypeStruct((NW, BINS * L), jnp.float32),
        ),
        mesh=mesh,
        compiler_params=pltpu.CompilerParams(needs_layout_passes=False),
        scratch_types=[
            pltpu.VMEM((CHUNK,), jnp.float32),
            pltpu.VMEM((CHUNK,), jnp.float32),
            pltpu.VMEM((UNROLL * TAB_STRIDE,), jnp.float32),
            pltpu.VMEM((UNROLL * TAB_STRIDE,), jnp.float32),
            pltpu.VMEM((BINS * L,), jnp.float32),
            pltpu.VMEM((BINS * L,), jnp.float32),
            pltpu.SemaphoreType.DMA,
            pltpu.SemaphoreType.DMA,
        ],
    )
    def hist_kernel(d_hbm, cnt_out, ls_out, d_v0, d_v1, cnt_tab, ls_tab,
                    cnt_fin, ls_fin, sem0, sem1):
        wid = lax.axis_index("s") * NC + lax.axis_index("c")
        zero16 = jnp.zeros((L,), jnp.float32)
        for k in range(UNROLL):
            for b in range(BINS):
                cnt_tab[pl.ds(k * TAB_STRIDE + b * L, L)] = zero16
                ls_tab[pl.ds(k * TAB_STRIDE + b * L, L)] = zero16

        lane = lax.iota(jnp.int32, L)
        lane_k = [lane + jnp.int32(k * TAB_STRIDE) for k in range(UNROLL)]
        ones16 = jnp.full((L,), 1.0, jnp.float32)
        mu2 = jnp.float32(MU * MU)
        base0 = wid * PER_W

        bufs = [d_v0, d_v1]
        sems = [sem0, sem1]

        def dma(c):
            return pltpu.make_async_copy(
                d_hbm.at[pl.ds(base0 + c * CHUNK, CHUNK)],
                bufs[c % 2], sems[c % 2])

        dma(0).start()
        for c in range(N_CHUNKS):
            dma(c).wait()
            if c + 1 < N_CHUNKS:
                dma(c + 1).start()
            d_v = bufs[c % 2]

            def vec_body(j, _):
                ds_ = []
                for k in range(UNROLL):
                    off = (j * UNROLL + k) * L
                    ds_.append(d_v[pl.ds(off, L)])
                addrs, losses = [], []
                for k in range(UNROLL):
                    d = ds_[k]
                    u = d * d
                    v = u + mu2
                    iv = lax.bitcast_convert_type(v, jnp.int32)
                    iv = jnp.int32(0x5F3759DF) - lax.shift_right_arithmetic(
                        iv, jnp.int32(1)
                    )
                    r = lax.bitcast_convert_type(iv, jnp.float32)
                    r = r * (jnp.float32(1.5)
                             - (jnp.float32(0.5) * v) * r * r)
                    s = v * r
                    losses.append(s - jnp.float32(MU))
                    g10 = jnp.abs(d) * r * jnp.float32(BINS)
                    bi = jnp.minimum(g10, jnp.float32(9.5)).astype(jnp.int32)
                    addrs.append((bi * L) | lane_k[k])
                for k in range(UNROLL):
                    plsc.addupdate_scatter(cnt_tab, [addrs[k]], ones16)
                    plsc.addupdate_scatter(ls_tab, [addrs[k]], losses[k])
                return _

            lax.fori_loop(0, VECS // UNROLL, vec_body, None)

        for b in range(BINS):
            crow = cnt_tab[pl.ds(b * L, L)]
            lrow = ls_tab[pl.ds(b * L, L)]
            for k in range(1, UNROLL):
                crow += cnt_tab[pl.ds(k * TAB_STRIDE + b * L, L)]
                lrow += ls_tab[pl.ds(k * TAB_STRIDE + b * L, L)]
            cnt_fin[pl.ds(b * L, L)] = crow
            ls_fin[pl.ds(b * L, L)] = lrow
        pltpu.sync_copy(cnt_fin, cnt_out.at[wid])
        pltpu.sync_copy(ls_fin, ls_out.at[wid])

    return hist_kernel(d_flat)


def _combine_kernel(cnt_ref, ls_ref, out_ref):
    n = jnp.float32(0.0)
    acc = jnp.float32(0.0)
    for b in range(BINS):
        cb = jnp.sum(cnt_ref[:, b * L:(b + 1) * L])
        sb = jnp.sum(ls_ref[:, b * L:(b + 1) * L])
        nz = cb > 0
        n += jnp.where(nz, 1.0, 0.0).astype(jnp.float32)
        acc += jnp.where(nz, sb / jnp.maximum(cb, 1.0), 0.0).astype(jnp.float32)
    out_ref[0, 0] = acc / jnp.maximum(n, 1.0)


def kernel(input, target):
    d = (input - target).reshape(-1)
    cnt, ls = _sc_histogram(d)
    res = pl.pallas_call(
        _combine_kernel,
        out_shape=jax.ShapeDtypeStruct((1, 1), jnp.float32),
        out_specs=pl.BlockSpec(memory_space=pltpu.MemorySpace.SMEM),
    )(cnt, ls)
    return res[0, 0]


# speedup vs baseline: 11.0342x; 11.0342x over previous
import functools

import jax
import jax.numpy as jnp
from jax import lax
from jax.experimental import pallas as pl
from jax.experimental.pallas import tpu as pltpu
from jax.experimental.pallas import tpu_sc as plsc

MU = 0.02
BINS = 10
NC = 2
NS = 16
L = 16
NW = NC * NS

N_ROWS = 2000000
ROWS_A = 7810 * 128
UNROLL = 5
TAB_STRIDE = 256


def _sc_histogram(d_flat, chunk, n_chunks):
    per_w = d_flat.shape[0] // NW
    assert per_w == chunk * n_chunks and (chunk // L) % UNROLL == 0
    mesh = plsc.VectorSubcoreMesh(
        core_axis_name="c", subcore_axis_name="s", num_cores=NC, num_subcores=NS
    )

    @functools.partial(
        pl.kernel,
        out_type=(
            jax.ShapeDtypeStruct((NW, BINS * L), jnp.float32),
            jax.ShapeDtypeStruct((NW, BINS * L), jnp.float32),
        ),
        mesh=mesh,
        compiler_params=pltpu.CompilerParams(needs_layout_passes=False),
        scratch_types=[
            pltpu.VMEM((chunk,), jnp.float32),
            pltpu.VMEM((chunk,), jnp.float32),
            pltpu.VMEM((UNROLL * TAB_STRIDE,), jnp.float32),
            pltpu.VMEM((UNROLL * TAB_STRIDE,), jnp.float32),
            pltpu.VMEM((BINS * L,), jnp.float32),
            pltpu.VMEM((BINS * L,), jnp.float32),
            pltpu.SemaphoreType.DMA,
            pltpu.SemaphoreType.DMA,
        ],
    )
    def hist_kernel(d_hbm, cnt_out, ls_out, d_v0, d_v1, cnt_tab, ls_tab,
                    cnt_fin, ls_fin, sem0, sem1):
        wid = lax.axis_index("s") * NC + lax.axis_index("c")
        zero16 = jnp.zeros((L,), jnp.float32)
        for k in range(UNROLL):
            for b in range(BINS):
                cnt_tab[pl.ds(k * TAB_STRIDE + b * L, L)] = zero16
                ls_tab[pl.ds(k * TAB_STRIDE + b * L, L)] = zero16

        lane = lax.iota(jnp.int32, L)
        lane_k = [lane + jnp.int32(k * TAB_STRIDE) for k in range(UNROLL)]
        ones16 = jnp.full((L,), 1.0, jnp.float32)
        mu2 = jnp.float32(MU * MU)
        base0 = wid * per_w

        bufs = [d_v0, d_v1]
        sems = [sem0, sem1]

        def dma(c):
            return pltpu.make_async_copy(
                d_hbm.at[pl.ds(base0 + c * chunk, chunk)],
                bufs[c % 2], sems[c % 2])

        dma(0).start()
        for c in range(n_chunks):
            dma(c).wait()
            if c + 1 < n_chunks:
                dma(c + 1).start()
            d_v = bufs[c % 2]

            def vec_body(j, _):
                ds_ = []
                for k in range(UNROLL):
                    off = (j * UNROLL + k) * L
                    ds_.append(d_v[pl.ds(off, L)])
                addrs, losses = [], []
                for k in range(UNROLL):
                    d = ds_[k]
                    u = d * d
                    v = u + mu2
                    iv = lax.bitcast_convert_type(v, jnp.int32)
                    iv = jnp.int32(0x5F3759DF) - lax.shift_right_arithmetic(
                        iv, jnp.int32(1)
                    )
                    r = lax.bitcast_convert_type(iv, jnp.float32)
                    r = r * (jnp.float32(1.5)
                             - (jnp.float32(0.5) * v) * r * r)
                    s = v * r
                    losses.append(s - jnp.float32(MU))
                    g10 = jnp.abs(d) * r * jnp.float32(BINS)
                    bi = jnp.minimum(g10, jnp.float32(9.5)).astype(jnp.int32)
                    addrs.append((bi * L) | lane_k[k])
                for k in range(UNROLL):
                    plsc.addupdate_scatter(cnt_tab, [addrs[k]], ones16)
                    plsc.addupdate_scatter(ls_tab, [addrs[k]], losses[k])
                return _

            lax.fori_loop(0, (chunk // L) // UNROLL, vec_body, None)

        for b in range(BINS):
            crow = cnt_tab[pl.ds(b * L, L)]
            lrow = ls_tab[pl.ds(b * L, L)]
            for k in range(1, UNROLL):
                crow += cnt_tab[pl.ds(k * TAB_STRIDE + b * L, L)]
                lrow += ls_tab[pl.ds(k * TAB_STRIDE + b * L, L)]
            cnt_fin[pl.ds(b * L, L)] = crow
            ls_fin[pl.ds(b * L, L)] = lrow
        pltpu.sync_copy(cnt_fin, cnt_out.at[wid])
        pltpu.sync_copy(ls_fin, ls_out.at[wid])

    return hist_kernel(d_flat)


def _combine_kernel(cnt_a, ls_a, cnt_b, ls_b, out_ref):
    n = jnp.float32(0.0)
    acc = jnp.float32(0.0)
    for b in range(BINS):
        sl = slice(b * L, (b + 1) * L)
        cb = jnp.sum(cnt_a[:, sl]) + jnp.sum(cnt_b[:, sl])
        sb = jnp.sum(ls_a[:, sl]) + jnp.sum(ls_b[:, sl])
        nz = cb > 0
        n += jnp.where(nz, 1.0, 0.0).astype(jnp.float32)
        acc += jnp.where(nz, sb / jnp.maximum(cb, 1.0), 0.0).astype(jnp.float32)
    out_ref[0, 0] = acc / jnp.maximum(n, 1.0)


def _phys_flat(x):
    return x.reshape(-1, 128, 4).transpose(0, 2, 1).reshape(-1)


def kernel(input, target):
    d_a = _phys_flat(input[:ROWS_A] - target[:ROWS_A])
    d_b = _phys_flat(input[ROWS_A:] - target[ROWS_A:])
    cnt_a, ls_a = _sc_histogram(d_a, 62480, 2)
    cnt_b, ls_b = _sc_histogram(d_b, 41680, 3)
    res = pl.pallas_call(
        _combine_kernel,
        out_shape=jax.ShapeDtypeStruct((1, 1), jnp.float32),
        out_specs=pl.BlockSpec(memory_space=pltpu.MemorySpace.SMEM),
    )(cnt_a, ls_a, cnt_b, ls_b)
    return res[0, 0]


# speedup vs baseline: 12.3251x; 1.1170x over previous
import functools

import jax
import jax.numpy as jnp
from jax import lax
from jax.experimental import pallas as pl
from jax.experimental.pallas import tpu as pltpu
from jax.experimental.pallas import tpu_sc as plsc

MU = 0.02
BINS = 10
NC = 2
NS = 16
L = 16
NW = NC * NS

N_ROWS = 2000000
ROWS_A = 7810 * 128
UNROLL = 5
TAB_STRIDE = 256


def _sc_histogram(d_flat, chunk, n_chunks):
    per_w = d_flat.shape[0] // NW
    assert per_w == chunk * n_chunks and (chunk // L) % UNROLL == 0
    mesh = plsc.VectorSubcoreMesh(
        core_axis_name="c", subcore_axis_name="s", num_cores=NC, num_subcores=NS
    )

    @functools.partial(
        pl.kernel,
        out_type=(
            jax.ShapeDtypeStruct((NW, BINS * L), jnp.float32),
            jax.ShapeDtypeStruct((NW, BINS * L), jnp.float32),
        ),
        mesh=mesh,
        compiler_params=pltpu.CompilerParams(needs_layout_passes=False),
        scratch_types=[
            pltpu.VMEM((chunk,), jnp.float32),
            pltpu.VMEM((chunk,), jnp.float32),
            pltpu.VMEM((UNROLL * TAB_STRIDE,), jnp.float32),
            pltpu.VMEM((UNROLL * TAB_STRIDE,), jnp.float32),
            pltpu.VMEM((BINS * L,), jnp.float32),
            pltpu.VMEM((BINS * L,), jnp.float32),
            pltpu.SemaphoreType.DMA,
            pltpu.SemaphoreType.DMA,
        ],
    )
    def hist_kernel(d_hbm, cnt_out, ls_out, d_v0, d_v1, cnt_tab, ls_tab,
                    cnt_fin, ls_fin, sem0, sem1):
        wid = lax.axis_index("s") * NC + lax.axis_index("c")
        zero16 = jnp.zeros((L,), jnp.float32)
        for k in range(UNROLL):
            for b in range(BINS):
                cnt_tab[pl.ds(k * TAB_STRIDE + b * L, L)] = zero16
                ls_tab[pl.ds(k * TAB_STRIDE + b * L, L)] = zero16

        lane = lax.iota(jnp.int32, L)
        lane_k = [lane + jnp.int32(k * TAB_STRIDE) for k in range(UNROLL)]
        ones16 = jnp.full((L,), 1.0, jnp.float32)
        mu2 = jnp.float32(MU * MU)
        base0 = wid * per_w

        bufs = [d_v0, d_v1]
        sems = [sem0, sem1]

        def dma(c):
            return pltpu.make_async_copy(
                d_hbm.at[pl.ds(base0 + c * chunk, chunk)],
                bufs[c % 2], sems[c % 2])

        dma(0).start()
        for c in range(n_chunks):
            dma(c).wait()
            if c + 1 < n_chunks:
                dma(c + 1).start()
            d_v = bufs[c % 2]

            def vec_body(j, _):
                ds_ = []
                for k in range(UNROLL):
                    off = (j * UNROLL + k) * L
                    ds_.append(d_v[pl.ds(off, L)])
                addrs, losses = [], []
                for k in range(UNROLL):
                    d = ds_[k]
                    u = d * d
                    v = u + mu2
                    iv = lax.bitcast_convert_type(v, jnp.int32)
                    iv = jnp.int32(0x5F3759DF) - lax.shift_right_arithmetic(
                        iv, jnp.int32(1)
                    )
                    r = lax.bitcast_convert_type(iv, jnp.float32)
                    r = r * (jnp.float32(1.5)
                             - (jnp.float32(0.5) * v) * r * r)
                    s = v * r
                    losses.append(s - jnp.float32(MU))
                    g10 = jnp.abs(d) * r * jnp.float32(BINS)
                    bi = jnp.minimum(g10, jnp.float32(9.5)).astype(jnp.int32)
                    addrs.append((bi * L) | lane_k[k])
                for k in range(UNROLL):
                    plsc.addupdate_scatter(cnt_tab, [addrs[k]], ones16)
                    plsc.addupdate_scatter(ls_tab, [addrs[k]], losses[k])
                return _

            lax.fori_loop(0, (chunk // L) // UNROLL, vec_body, None)

        for b in range(BINS):
            crow = cnt_tab[pl.ds(b * L, L)]
            lrow = ls_tab[pl.ds(b * L, L)]
            for k in range(1, UNROLL):
                crow += cnt_tab[pl.ds(k * TAB_STRIDE + b * L, L)]
                lrow += ls_tab[pl.ds(k * TAB_STRIDE + b * L, L)]
            cnt_fin[pl.ds(b * L, L)] = crow
            ls_fin[pl.ds(b * L, L)] = lrow
        pltpu.sync_copy(cnt_fin, cnt_out.at[wid])
        pltpu.sync_copy(ls_fin, ls_out.at[wid])

    return hist_kernel(d_flat)


def _combine_kernel(*refs):
    out_ref = refs[-1]
    cnts = refs[0:-1:2]
    lss = refs[1:-1:2]
    n = jnp.float32(0.0)
    acc = jnp.float32(0.0)
    for b in range(BINS):
        sl = slice(b * L, (b + 1) * L)
        cb = sum(jnp.sum(c[:, sl]) for c in cnts)
        sb = sum(jnp.sum(s[:, sl]) for s in lss)
        nz = cb > 0
        n += jnp.where(nz, 1.0, 0.0).astype(jnp.float32)
        acc += jnp.where(nz, sb / jnp.maximum(cb, 1.0), 0.0).astype(jnp.float32)
    out_ref[0, 0] = acc / jnp.maximum(n, 1.0)


def _phys_flat(x):
    return x.reshape(-1, 128, 4).transpose(0, 2, 1).reshape(-1)


def kernel(input, target):
    bounds = [0, 3905 * 128, 7810 * 128, 11715 * 128, N_ROWS]
    cfgs = [(5680, 11), (5680, 11), (5680, 11), (31280, 2)]
    parts = []
    for i in range(4):
        lo, hi = bounds[i], bounds[i + 1]
        d_i = _phys_flat(input[lo:hi] - target[lo:hi])
        parts.extend(_sc_histogram(d_i, *cfgs[i]))
    res = pl.pallas_call(
        _combine_kernel,
        out_shape=jax.ShapeDtypeStruct((1, 1), jnp.float32),
        out_specs=pl.BlockSpec(memory_space=pltpu.MemorySpace.SMEM),
    )(*parts)
    return res[0, 0]
